# baseline (speedup 1.0000x reference)
import functools

import jax
import jax.numpy as jnp
import numpy as np
from jax import lax
from jax.experimental import pallas as pl
from jax.experimental.pallas import tpu as pltpu
from jax.experimental.pallas import tpu_sc as plsc

C = 19
N, H, W = 8, 512, 512
TOTAL = N * H * W
K_KEPT = 100000
THRESH_V = float(-np.log(np.float32(0.7)))

NB = 4096
L = 16
NC, NS = 2, 16
NW = NC * NS
PER_W = TOTAL // NW
CHUNK = 8192
UNROLL = 4
HR, HL = 32, 128
HCH = NB // L


RB = 512


def _nll_body(x_ref, lab_ref, out_ref):
    x = x_ref[0]
    lab = lab_ref[0]
    m = jnp.max(x, axis=0)
    s = jnp.sum(jnp.exp(x - m[None, :, :]), axis=0)
    lse = jnp.log(s) + m
    sel = jnp.zeros_like(m)
    for c in range(C):
        sel = sel + jnp.where(lab == c, x[c], 0.0)
    out_ref[...] = jnp.maximum(lse - sel, 0.0)


def _nll_part(logits, labels, n0, nn):
    return pl.pallas_call(
        _nll_body,
        grid=(nn, H // RB),
        in_specs=[
            pl.BlockSpec((1, C, RB, W), lambda n, h: (n + n0, 0, h, 0)),
            pl.BlockSpec((1, RB, W), lambda n, h: (n + n0, h, 0)),
        ],
        out_specs=pl.BlockSpec((RB, W), lambda n, h: (n * (H // RB) + h, 0)),
        out_shape=jax.ShapeDtypeStruct((nn * H, W), jnp.float32),
    )(logits, labels)


def _hchunk(ref, jr):
    r = lax.shift_right_logical(jr, 3)
    c = jnp.bitwise_and(jr, 7)
    return ref[r, pl.ds(c * L, L)]


def _rev_cumsum(vals):
    r = lax.rev(vals, (0,))
    return r, plsc.cumsum(r)


def _lane_pick(vec, lane):
    pos = lax.iota(jnp.int32, L)
    zero = jnp.zeros_like(vec)
    return jnp.sum(jnp.where(pos == lane, vec, zero), axis=0)


def _scan_pivot(hc, hs, cum0, sum0, target):
    pos = lax.iota(jnp.int32, L)

    def body(j, carry):
        cum, csum, found, p, cab, sab, bc, bs = carry
        jr = HCH - 1 - j
        c, cs = _rev_cumsum(_hchunk(hc, jr))
        tot = _lane_pick(cs, L - 1)
        within = (cum + cs) >= target
        hit = jnp.logical_and(found == 0, cum + tot >= target)
        first = jnp.min(jnp.where(within, pos, jnp.full((L,), L, jnp.int32)), axis=0)
        first = jnp.minimum(first, L - 1)
        p_new = jr * L + (L - 1) - first
        cs_f = _lane_pick(cs, first)
        c_f = _lane_pick(c, first)
        p = jnp.where(hit, p_new, p)
        cab = jnp.where(hit, cum + cs_f - c_f, cab)
        bc = jnp.where(hit, c_f, bc)
        if hs is not None:
            s, ss = _rev_cumsum(_hchunk(hs, jr))
            stot = _lane_pick(ss, L - 1)
            ss_f = _lane_pick(ss, first)
            s_f = _lane_pick(s, first)
            sab = jnp.where(hit, csum + ss_f - s_f, sab)
            bs = jnp.where(hit, s_f, bs)
        else:
            stot = jnp.asarray(0.0, jnp.float32)
        found = jnp.where(hit, jnp.asarray(1, jnp.int32), found)
        keep = found == 0
        cum = jnp.where(keep, cum + tot, cum)
        csum = jnp.where(keep, csum + stot, csum)
        return (cum, csum, found, p, cab, sab, bc, bs)

    init = (cum0, sum0, jnp.asarray(0, jnp.int32), jnp.asarray(0, jnp.int32),
            jnp.asarray(0, jnp.int32), jnp.asarray(0.0, jnp.float32),
            jnp.asarray(1, jnp.int32), jnp.asarray(0.0, jnp.float32))
    out = lax.fori_loop(0, HCH, body, init)
    return out[3], out[4], out[5], out[6], out[7]


def _zero_hist(ref, dtype):
    z = jnp.zeros((L,), dtype)

    def zbody(j, _):
        r = lax.shift_right_logical(j, 3)
        c = jnp.bitwise_and(j, 7)
        ref[r, pl.ds(c * L, L)] = z
        return 0

    lax.fori_loop(0, HCH, zbody, 0)


def _fill_row_indices(idx):
    pos = lax.iota(jnp.int32, L)

    def ibody(i, _):
        idx[pl.ds(i * L, L)] = pos + i * L
        return 0

    lax.fori_loop(0, HR // L, ibody, 0)


def _spmem_reduce(my2d, shared, idx, sid):
    @pl.when(sid == 0)
    def _():
        pltpu.sync_copy(my2d, shared)

    plsc.subcore_barrier()

    @pl.when(sid != 0)
    def _():
        pltpu.sync_copy(my2d, shared.at[idx], add=True)

    plsc.subcore_barrier()


def _load_combined(src_hbm, a, b, accumulate=False):
    if not accumulate:
        pltpu.sync_copy(src_hbm.at[0], a)
        pltpu.sync_copy(src_hbm.at[1], b)

        def cbody(j, _):
            r = lax.shift_right_logical(j, 3)
            sl = pl.ds(jnp.bitwise_and(j, 7) * L, L)
            a[r, sl] = a[r, sl] + b[r, sl]
            return 0

        lax.fori_loop(0, HCH, cbody, 0)
    else:
        for part in (0, 1):
            pltpu.sync_copy(src_hbm.at[part], b)

            def cbody(j, _):
                r = lax.shift_right_logical(j, 3)
                sl = pl.ds(jnp.bitwise_and(j, 7) * L, L)
                a[r, sl] = a[r, sl] + b[r, sl]
                return 0

            lax.fori_loop(0, HCH, cbody, 0)


def _hist1_body(nch, nll_hbm, cnt_hbm, scal_hbm, buf, hcnt, svec, idx, shc):
    cid = lax.axis_index("c")
    sid = lax.axis_index("s")
    wid = sid * NC + cid
    base = wid * (nch * CHUNK)
    ones = jnp.ones((L,), jnp.int32)
    _zero_hist(hcnt, jnp.int32)
    _fill_row_indices(idx)
    zf = jnp.zeros((L,), jnp.float32)

    def chunk_body(ci, carry):
        pltpu.sync_copy(nll_hbm.at[pl.ds(base + ci * CHUNK, CHUNK)], buf)

        def vbody(i, c2):
            tc, ts = c2
            for k in range(UNROLL):
                v = buf[pl.ds((i * UNROLL + k) * L, L)]
                bits = lax.bitcast_convert_type(v, jnp.int32)
                b1 = lax.shift_right_logical(bits, 19)
                plsc.addupdate_scatter(
                    hcnt, [lax.shift_right_logical(b1, 7),
                           jnp.bitwise_and(b1, HL - 1)], ones)
                mk = v > THRESH_V
                tc = tc + jnp.where(mk, 1.0, 0.0)
                ts = ts + jnp.where(mk, v, 0.0)
            return (tc, ts)

        return lax.fori_loop(0, CHUNK // L // UNROLL, vbody, carry)

    tc, ts = lax.fori_loop(0, nch, chunk_body, (zf, zf))
    svec[pl.ds(0, L)] = tc
    svec[pl.ds(L, L)] = ts
    pltpu.sync_copy(svec, scal_hbm.at[pl.ds(wid * 2 * L, 2 * L)])
    _spmem_reduce(hcnt, shc, idx, sid)

    @pl.when(sid == 0)
    def _():
        pltpu.sync_copy(shc, cnt_hbm.at[cid])


def _hist2_body(ncha, nchb, nlla_hbm, nllb_hbm, cnt1a_hbm, cnt1b_hbm,
                cnt2_hbm, sum2_hbm, scal_hbm,
                buf, rc, rb, hcnt, hsum, svec, idx, shc, shs):
    cid = lax.axis_index("c")
    sid = lax.axis_index("s")
    wid = sid * NC + cid
    ones = jnp.ones((L,), jnp.int32)
    _fill_row_indices(idx)
    _load_combined(cnt1a_hbm, rc, rb)
    _load_combined(cnt1b_hbm, rc, rb, accumulate=True)
    p1, _, _, _, _ = _scan_pivot(rc, None, jnp.asarray(0, jnp.int32),
                                 jnp.asarray(0.0, jnp.float32), K_KEPT)
    _zero_hist(hcnt, jnp.int32)
    _zero_hist(hsum, jnp.float32)
    zf = jnp.zeros((L,), jnp.float32)

    def make_chunk_body(nll_hbm, nch):
      base = wid * (nch * CHUNK)

      def chunk_body(ci, carry):
        pltpu.sync_copy(nll_hbm.at[pl.ds(base + ci * CHUNK, CHUNK)], buf)

        def vbody(i, sacc):
            for k in range(UNROLL):
                v = buf[pl.ds((i * UNROLL + k) * L, L)]
                bits = lax.bitcast_convert_type(v, jnp.int32)
                b1 = lax.shift_right_logical(bits, 19)
                mk = b1 == p1
                anyhit = jnp.sum(jnp.where(mk, 1, 0), axis=0) > 0

                @pl.when(anyhit)
                def _(v=v, bits=bits, mk=mk):
                    b2 = jnp.bitwise_and(lax.shift_right_logical(bits, 7), NB - 1)
                    r2 = lax.shift_right_logical(b2, 7)
                    l2 = jnp.bitwise_and(b2, HL - 1)
                    plsc.addupdate_scatter(hcnt, [r2, l2], ones, mask=mk)
                    plsc.addupdate_scatter(hsum, [r2, l2], v, mask=mk)

                sacc = sacc + jnp.where(b1 > p1, v, 0.0)
            return sacc

        return lax.fori_loop(0, CHUNK // L // UNROLL, vbody, carry)

      return chunk_body

    sacc = lax.fori_loop(0, ncha, make_chunk_body(nlla_hbm, ncha), zf)
    sacc = lax.fori_loop(0, nchb, make_chunk_body(nllb_hbm, nchb), sacc)
    svec[pl.ds(0, L)] = sacc
    pltpu.sync_copy(svec, scal_hbm.at[pl.ds(wid * L, L)])
    _spmem_reduce(hcnt, shc, idx, sid)

    @pl.when(sid == 0)
    def _():
        pltpu.sync_copy(shc, cnt2_hbm.at[cid])

    _spmem_reduce(hsum, shs, idx, sid)

    @pl.when(sid == 0)
    def _():
        pltpu.sync_copy(shs, sum2_hbm.at[cid])


def _final_body(cnt1a_hbm, cnt1b_hbm, cnt2_hbm, sum2_hbm, scal1_hbm,
                scal2_hbm, out_hbm, rc1, rc2, rs2, rbi, rbf, sc1, sc2, ov):
    cid = lax.axis_index("c")
    sid = lax.axis_index("s")

    @pl.when(jnp.logical_and(cid == 0, sid == 0))
    def _():
        _load_combined(cnt1a_hbm, rc1, rbi)
        _load_combined(cnt1b_hbm, rc1, rbi, accumulate=True)
        _load_combined(cnt2_hbm, rc2, rbi)
        _load_combined(sum2_hbm, rs2, rbf)
        pltpu.sync_copy(scal1_hbm, sc1)
        pltpu.sync_copy(scal2_hbm, sc2)

        def sbody(r, carry):
            a, b, s1 = carry
            return (a + sc1[pl.ds(r * 2 * L, L)],
                    b + sc1[pl.ds(r * 2 * L + L, L)], s1)

        zf = jnp.zeros((L,), jnp.float32)
        acc_c, acc_s, _ = lax.fori_loop(0, 2 * NW, sbody, (zf, zf, zf))

        def s2body(r, carry):
            return carry + sc2[pl.ds(r * L, L)]

        acc_s1 = lax.fori_loop(0, NW, s2body, zf)

        def splat(x):
            return lax.broadcast_in_dim(x, (L,), ())

        cnt_gt = splat(jnp.sum(acc_c, axis=0))
        sum_gt = splat(jnp.sum(acc_s, axis=0))
        sum_above1 = jnp.sum(acc_s1, axis=0)
        mean_a = sum_gt / jnp.maximum(cnt_gt, 1.0)
        cond = cnt_gt > float(K_KEPT)

        _, cab1, _, _, _ = _scan_pivot(
            rc1, None, jnp.asarray(0, jnp.int32), jnp.asarray(0.0, jnp.float32),
            K_KEPT)
        _, cab2, sab2, bc2, bs2 = _scan_pivot(rc2, rs2, cab1, sum_above1,
                                              K_KEPT)
        needed = splat((K_KEPT - cab2).astype(jnp.float32))
        bin_avg = splat(bs2) / jnp.maximum(splat(bc2.astype(jnp.float32)), 1.0)
        mean_b = (splat(sab2) + needed * bin_avg) * (1.0 / float(K_KEPT))

        ov[pl.ds(0, L)] = jnp.where(cond, mean_a, mean_b)
        pltpu.sync_copy(ov, out_hbm)


NIMG_A = 5
NIMG_B = N - NIMG_A
NCH_A = NIMG_A * H * W // NW // CHUNK
NCH_B = NIMG_B * H * W // NW // CHUNK


@functools.lru_cache(maxsize=1)
def _sc_kernels():
    mesh = plsc.VectorSubcoreMesh(core_axis_name="c", subcore_axis_name="s")
    f32, i32 = jnp.float32, jnp.int32
    cp = pltpu.CompilerParams(needs_layout_passes=False)

    def make_hist1(nch):
        return pl.kernel(
            functools.partial(_hist1_body, nch),
            out_type=(jax.ShapeDtypeStruct((NC, HR, HL), i32),
                      jax.ShapeDtypeStruct((NW * 2 * L,), f32)),
            mesh=mesh,
            compiler_params=cp,
            scratch_types=[pltpu.VMEM((CHUNK,), f32),
                           pltpu.VMEM((HR, HL), i32),
                           pltpu.VMEM((2 * L,), f32),
                           pltpu.VMEM((HR,), i32),
                           pltpu.VMEM_SHARED((HR, HL), i32)],
        )

    hist1a = make_hist1(NCH_A)
    hist1b = make_hist1(NCH_B)
    hist2 = pl.kernel(
        functools.partial(_hist2_body, NCH_A, NCH_B),
        out_type=(jax.ShapeDtypeStruct((NC, HR, HL), i32),
                  jax.ShapeDtypeStruct((NC, HR, HL), f32),
                  jax.ShapeDtypeStruct((NW * L,), f32)),
        mesh=mesh,
        compiler_params=cp,
        scratch_types=[pltpu.VMEM((CHUNK,), f32),
                       pltpu.VMEM((HR, HL), i32),
                       pltpu.VMEM((HR, HL), i32),
                       pltpu.VMEM((HR, HL), i32),
                       pltpu.VMEM((HR, HL), f32),
                       pltpu.VMEM((L,), f32),
                       pltpu.VMEM((HR,), i32),
                       pltpu.VMEM_SHARED((HR, HL), i32),
                       pltpu.VMEM_SHARED((HR, HL), f32)],
    )
    final = pl.kernel(
        _final_body,
        out_type=jax.ShapeDtypeStruct((L,), f32),
        mesh=mesh,
        compiler_params=cp,
        scratch_types=[pltpu.VMEM((HR, HL), i32),
                       pltpu.VMEM((HR, HL), i32),
                       pltpu.VMEM((HR, HL), f32),
                       pltpu.VMEM((HR, HL), i32),
                       pltpu.VMEM((HR, HL), f32),
                       pltpu.VMEM((2 * NW * 2 * L,), f32),
                       pltpu.VMEM((NW * L,), f32),
                       pltpu.VMEM((L,), f32)],
    )
    return hist1a, hist1b, hist2, final


def kernel(logits, labels):
    hist1a, hist1b, hist2, final = _sc_kernels()
    labels = labels.astype(jnp.int32)
    nll_a = _nll_part(logits, labels, 0, NIMG_A).reshape(-1)
    cnt1a, scal1a = hist1a(nll_a)
    nll_b = _nll_part(logits, labels, NIMG_A, NIMG_B).reshape(-1)
    cnt1b, scal1b = hist1b(nll_b)
    scal1 = jnp.concatenate([scal1a, scal1b])
    cnt2, sum2, scal2 = hist2(nll_a, nll_b, cnt1a, cnt1b)
    out = final(cnt1a, cnt1b, cnt2, sum2, scal1, scal2)
    return out[0]


# speedup vs baseline: 1.3127x; 1.3127x over previous
import functools

import jax
import jax.numpy as jnp
import numpy as np
from jax import lax
from jax.experimental import pallas as pl
from jax.experimental.pallas import tpu as pltpu
from jax.experimental.pallas import tpu_sc as plsc

C = 19
N, H, W = 8, 512, 512
TOTAL = N * H * W
K_KEPT = 100000
THRESH_V = float(-np.log(np.float32(0.7)))

NB = 4096
L = 16
NC, NS = 2, 16
NW = NC * NS
PER_W = TOTAL // NW
CHUNK = 8192
UNROLL = 4
HR, HL = 32, 128
HCH = NB // L


RB = 512


def _nll_body(x_ref, lab_ref, out_ref):
    x = x_ref[0]
    lab = lab_ref[0]
    m = jnp.max(x, axis=0)
    s = jnp.sum(jnp.exp(x - m[None, :, :]), axis=0)
    lse = jnp.log(s) + m
    sel = jnp.zeros_like(m)
    for c in range(C):
        sel = sel + jnp.where(lab == c, x[c], 0.0)
    out_ref[...] = jnp.maximum(lse - sel, 0.0)


def _nll_part(logits, labels, n0, nn):
    return pl.pallas_call(
        _nll_body,
        grid=(nn, H // RB),
        in_specs=[
            pl.BlockSpec((1, C, RB, W), lambda n, h: (n + n0, 0, h, 0)),
            pl.BlockSpec((1, RB, W), lambda n, h: (n + n0, h, 0)),
        ],
        out_specs=pl.BlockSpec((RB, W), lambda n, h: (n * (H // RB) + h, 0)),
        out_shape=jax.ShapeDtypeStruct((nn * H, W), jnp.float32),
    )(logits, labels)


def _hchunk(ref, jr):
    r = lax.shift_right_logical(jr, 3)
    c = jnp.bitwise_and(jr, 7)
    return ref[r, pl.ds(c * L, L)]


def _rev_cumsum(vals):
    r = lax.rev(vals, (0,))
    return r, plsc.cumsum(r)


def _lane_pick(vec, lane):
    pos = lax.iota(jnp.int32, L)
    zero = jnp.zeros_like(vec)
    return jnp.sum(jnp.where(pos == lane, vec, zero), axis=0)


def _scan_pivot(hc, hs, cum0, sum0, target):
    pos = lax.iota(jnp.int32, L)

    def body(j, carry):
        cum, csum, found, p, cab, sab, bc, bs = carry
        jr = HCH - 1 - j
        c, cs = _rev_cumsum(_hchunk(hc, jr))
        tot = _lane_pick(cs, L - 1)
        within = (cum + cs) >= target
        hit = jnp.logical_and(found == 0, cum + tot >= target)
        first = jnp.min(jnp.where(within, pos, jnp.full((L,), L, jnp.int32)), axis=0)
        first = jnp.minimum(first, L - 1)
        p_new = jr * L + (L - 1) - first
        cs_f = _lane_pick(cs, first)
        c_f = _lane_pick(c, first)
        p = jnp.where(hit, p_new, p)
        cab = jnp.where(hit, cum + cs_f - c_f, cab)
        bc = jnp.where(hit, c_f, bc)
        if hs is not None:
            s, ss = _rev_cumsum(_hchunk(hs, jr))
            stot = _lane_pick(ss, L - 1)
            ss_f = _lane_pick(ss, first)
            s_f = _lane_pick(s, first)
            sab = jnp.where(hit, csum + ss_f - s_f, sab)
            bs = jnp.where(hit, s_f, bs)
        else:
            stot = jnp.asarray(0.0, jnp.float32)
        found = jnp.where(hit, jnp.asarray(1, jnp.int32), found)
        keep = found == 0
        cum = jnp.where(keep, cum + tot, cum)
        csum = jnp.where(keep, csum + stot, csum)
        return (cum, csum, found, p, cab, sab, bc, bs)

    init = (cum0, sum0, jnp.asarray(0, jnp.int32), jnp.asarray(0, jnp.int32),
            jnp.asarray(0, jnp.int32), jnp.asarray(0.0, jnp.float32),
            jnp.asarray(1, jnp.int32), jnp.asarray(0.0, jnp.float32))
    out = lax.fori_loop(0, HCH, body, init)
    return out[3], out[4], out[5], out[6], out[7]


def _zero_hist(ref, dtype):
    z = jnp.zeros((L,), dtype)

    def zbody(j, _):
        r = lax.shift_right_logical(j, 3)
        c = jnp.bitwise_and(j, 7)
        ref[r, pl.ds(c * L, L)] = z
        return 0

    lax.fori_loop(0, HCH, zbody, 0)


def _fill_row_indices(idx):
    pos = lax.iota(jnp.int32, L)

    def ibody(i, _):
        idx[pl.ds(i * L, L)] = pos + i * L
        return 0

    lax.fori_loop(0, HR // L, ibody, 0)


def _spmem_reduce(my2d, shared, idx, sid):
    @pl.when(sid == 0)
    def _():
        pltpu.sync_copy(my2d, shared)

    plsc.subcore_barrier()

    @pl.when(sid != 0)
    def _():
        pltpu.sync_copy(my2d, shared.at[idx], add=True)

    plsc.subcore_barrier()


def _load_combined(src_hbm, a, b, accumulate=False):
    if not accumulate:
        pltpu.sync_copy(src_hbm.at[0], a)
        pltpu.sync_copy(src_hbm.at[1], b)

        def cbody(j, _):
            r = lax.shift_right_logical(j, 3)
            sl = pl.ds(jnp.bitwise_and(j, 7) * L, L)
            a[r, sl] = a[r, sl] + b[r, sl]
            return 0

        lax.fori_loop(0, HCH, cbody, 0)
    else:
        for part in (0, 1):
            pltpu.sync_copy(src_hbm.at[part], b)

            def cbody(j, _):
                r = lax.shift_right_logical(j, 3)
                sl = pl.ds(jnp.bitwise_and(j, 7) * L, L)
                a[r, sl] = a[r, sl] + b[r, sl]
                return 0

            lax.fori_loop(0, HCH, cbody, 0)


def _hist1_body(nch, nll_hbm, cnt_hbm, scal_hbm, buf, hcnt, svec, idx, shc):
    cid = lax.axis_index("c")
    sid = lax.axis_index("s")
    wid = sid * NC + cid
    base = wid * (nch * CHUNK)
    ones = jnp.ones((L,), jnp.int32)
    _zero_hist(hcnt, jnp.int32)
    _fill_row_indices(idx)
    zf = jnp.zeros((L,), jnp.float32)

    def chunk_body(ci, carry):
        pltpu.sync_copy(nll_hbm.at[pl.ds(base + ci * CHUNK, CHUNK)], buf)

        def vbody(i, c2):
            tc, ts = c2
            for k in range(UNROLL):
                v = buf[pl.ds((i * UNROLL + k) * L, L)]
                bits = lax.bitcast_convert_type(v, jnp.int32)
                b1 = lax.shift_right_logical(bits, 19)
                plsc.addupdate_scatter(
                    hcnt, [lax.shift_right_logical(b1, 7),
                           jnp.bitwise_and(b1, HL - 1)], ones)
                mk = v > THRESH_V
                tc = tc + jnp.where(mk, 1.0, 0.0)
                ts = ts + jnp.where(mk, v, 0.0)
            return (tc, ts)

        return lax.fori_loop(0, CHUNK // L // UNROLL, vbody, carry)

    tc, ts = lax.fori_loop(0, nch, chunk_body, (zf, zf))
    svec[pl.ds(0, L)] = tc
    svec[pl.ds(L, L)] = ts
    pltpu.sync_copy(svec, scal_hbm.at[pl.ds(wid * 2 * L, 2 * L)])
    _spmem_reduce(hcnt, shc, idx, sid)

    @pl.when(sid == 0)
    def _():
        pltpu.sync_copy(shc, cnt_hbm.at[cid])


def _hist2_body(ncha, nchb, nlla_hbm, nllb_hbm, cnt1a_hbm, cnt1b_hbm,
                cnt2_hbm, sum2_hbm, scal_hbm,
                buf, rc, rb, hcnt, hsum, svec, idx, shc, shs):
    cid = lax.axis_index("c")
    sid = lax.axis_index("s")
    wid = sid * NC + cid
    ones = jnp.ones((L,), jnp.int32)
    _fill_row_indices(idx)
    _load_combined(cnt1a_hbm, rc, rb)
    _load_combined(cnt1b_hbm, rc, rb, accumulate=True)
    p1, _, _, _, _ = _scan_pivot(rc, None, jnp.asarray(0, jnp.int32),
                                 jnp.asarray(0.0, jnp.float32), K_KEPT)
    _zero_hist(hcnt, jnp.int32)
    _zero_hist(hsum, jnp.float32)
    zf = jnp.zeros((L,), jnp.float32)

    def make_chunk_body(nll_hbm, nch):
      base = wid * (nch * CHUNK)

      def chunk_body(ci, carry):
        pltpu.sync_copy(nll_hbm.at[pl.ds(base + ci * CHUNK, CHUNK)], buf)

        def vbody(i, sacc):
            for k in range(UNROLL):
                v = buf[pl.ds((i * UNROLL + k) * L, L)]
                bits = lax.bitcast_convert_type(v, jnp.int32)
                b1 = lax.shift_right_logical(bits, 19)
                mk = b1 == p1
                b2 = jnp.bitwise_and(lax.shift_right_logical(bits, 7), NB - 1)
                r2 = lax.shift_right_logical(b2, 7)
                l2 = jnp.bitwise_and(b2, HL - 1)
                plsc.addupdate_scatter(hcnt, [r2, l2], ones, mask=mk)
                plsc.addupdate_scatter(hsum, [r2, l2], v, mask=mk)
                sacc = sacc + jnp.where(b1 > p1, v, 0.0)
            return sacc

        return lax.fori_loop(0, CHUNK // L // UNROLL, vbody, carry)

      return chunk_body

    sacc = lax.fori_loop(0, ncha, make_chunk_body(nlla_hbm, ncha), zf)
    sacc = lax.fori_loop(0, nchb, make_chunk_body(nllb_hbm, nchb), sacc)
    svec[pl.ds(0, L)] = sacc
    pltpu.sync_copy(svec, scal_hbm.at[pl.ds(wid * L, L)])
    _spmem_reduce(hcnt, shc, idx, sid)

    @pl.when(sid == 0)
    def _():
        pltpu.sync_copy(shc, cnt2_hbm.at[cid])

    _spmem_reduce(hsum, shs, idx, sid)

    @pl.when(sid == 0)
    def _():
        pltpu.sync_copy(shs, sum2_hbm.at[cid])


def _final_body(cnt1a_hbm, cnt1b_hbm, cnt2_hbm, sum2_hbm, scal1_hbm,
                scal2_hbm, out_hbm, rc1, rc2, rs2, rbi, rbf, sc1, sc2, ov):
    cid = lax.axis_index("c")
    sid = lax.axis_index("s")

    @pl.when(jnp.logical_and(cid == 0, sid == 0))
    def _():
        _load_combined(cnt1a_hbm, rc1, rbi)
        _load_combined(cnt1b_hbm, rc1, rbi, accumulate=True)
        _load_combined(cnt2_hbm, rc2, rbi)
        _load_combined(sum2_hbm, rs2, rbf)
        pltpu.sync_copy(scal1_hbm, sc1)
        pltpu.sync_copy(scal2_hbm, sc2)

        def sbody(r, carry):
            a, b, s1 = carry
            return (a + sc1[pl.ds(r * 2 * L, L)],
                    b + sc1[pl.ds(r * 2 * L + L, L)], s1)

        zf = jnp.zeros((L,), jnp.float32)
        acc_c, acc_s, _ = lax.fori_loop(0, 2 * NW, sbody, (zf, zf, zf))

        def s2body(r, carry):
            return carry + sc2[pl.ds(r * L, L)]

        acc_s1 = lax.fori_loop(0, NW, s2body, zf)

        def splat(x):
            return lax.broadcast_in_dim(x, (L,), ())

        cnt_gt = splat(jnp.sum(acc_c, axis=0))
        sum_gt = splat(jnp.sum(acc_s, axis=0))
        sum_above1 = jnp.sum(acc_s1, axis=0)
        mean_a = sum_gt / jnp.maximum(cnt_gt, 1.0)
        cond = cnt_gt > float(K_KEPT)

        _, cab1, _, _, _ = _scan_pivot(
            rc1, None, jnp.asarray(0, jnp.int32), jnp.asarray(0.0, jnp.float32),
            K_KEPT)
        _, cab2, sab2, bc2, bs2 = _scan_pivot(rc2, rs2, cab1, sum_above1,
                                              K_KEPT)
        needed = splat((K_KEPT - cab2).astype(jnp.float32))
        bin_avg = splat(bs2) / jnp.maximum(splat(bc2.astype(jnp.float32)), 1.0)
        mean_b = (splat(sab2) + needed * bin_avg) * (1.0 / float(K_KEPT))

        ov[pl.ds(0, L)] = jnp.where(cond, mean_a, mean_b)
        pltpu.sync_copy(ov, out_hbm)


NIMG_A = 5
NIMG_B = N - NIMG_A
NCH_A = NIMG_A * H * W // NW // CHUNK
NCH_B = NIMG_B * H * W // NW // CHUNK


@functools.lru_cache(maxsize=1)
def _sc_kernels():
    mesh = plsc.VectorSubcoreMesh(core_axis_name="c", subcore_axis_name="s")
    f32, i32 = jnp.float32, jnp.int32
    cp = pltpu.CompilerParams(needs_layout_passes=False)

    def make_hist1(nch):
        return pl.kernel(
            functools.partial(_hist1_body, nch),
            out_type=(jax.ShapeDtypeStruct((NC, HR, HL), i32),
                      jax.ShapeDtypeStruct((NW * 2 * L,), f32)),
            mesh=mesh,
            compiler_params=cp,
            scratch_types=[pltpu.VMEM((CHUNK,), f32),
                           pltpu.VMEM((HR, HL), i32),
                           pltpu.VMEM((2 * L,), f32),
                           pltpu.VMEM((HR,), i32),
                           pltpu.VMEM_SHARED((HR, HL), i32)],
        )

    hist1a = make_hist1(NCH_A)
    hist1b = make_hist1(NCH_B)
    hist2 = pl.kernel(
        functools.partial(_hist2_body, NCH_A, NCH_B),
        out_type=(jax.ShapeDtypeStruct((NC, HR, HL), i32),
                  jax.ShapeDtypeStruct((NC, HR, HL), f32),
                  jax.ShapeDtypeStruct((NW * L,), f32)),
        mesh=mesh,
        compiler_params=cp,
        scratch_types=[pltpu.VMEM((CHUNK,), f32),
                       pltpu.VMEM((HR, HL), i32),
                       pltpu.VMEM((HR, HL), i32),
                       pltpu.VMEM((HR, HL), i32),
                       pltpu.VMEM((HR, HL), f32),
                       pltpu.VMEM((L,), f32),
                       pltpu.VMEM((HR,), i32),
                       pltpu.VMEM_SHARED((HR, HL), i32),
                       pltpu.VMEM_SHARED((HR, HL), f32)],
    )
    final = pl.kernel(
        _final_body,
        out_type=jax.ShapeDtypeStruct((L,), f32),
        mesh=mesh,
        compiler_params=cp,
        scratch_types=[pltpu.VMEM((HR, HL), i32),
                       pltpu.VMEM((HR, HL), i32),
                       pltpu.VMEM((HR, HL), f32),
                       pltpu.VMEM((HR, HL), i32),
                       pltpu.VMEM((HR, HL), f32),
                       pltpu.VMEM((2 * NW * 2 * L,), f32),
                       pltpu.VMEM((NW * L,), f32),
                       pltpu.VMEM((L,), f32)],
    )
    return hist1a, hist1b, hist2, final


def kernel(logits, labels):
    hist1a, hist1b, hist2, final = _sc_kernels()
    labels = labels.astype(jnp.int32)
    nll_a = _nll_part(logits, labels, 0, NIMG_A).reshape(-1)
    cnt1a, scal1a = hist1a(nll_a)
    nll_b = _nll_part(logits, labels, NIMG_A, NIMG_B).reshape(-1)
    cnt1b, scal1b = hist1b(nll_b)
    scal1 = jnp.concatenate([scal1a, scal1b])
    cnt2, sum2, scal2 = hist2(nll_a, nll_b, cnt1a, cnt1b)
    out = final(cnt1a, cnt1b, cnt2, sum2, scal1, scal2)
    return out[0]
